# baseline (speedup 1.0000x reference)
import functools

import jax
import jax.numpy as jnp
from jax import lax
from jax.experimental import pallas as pl
from jax.experimental.pallas import tpu as pltpu
from jax.experimental.pallas import tpu_sc as plsc

_B = 16384
_K = 64
_SEG = 31256
_HALF = _SEG * 16
_SPLIT = 499904
_VCAP = _B + 128
_LCAP = _B + 16
_PHASES = _K * 2


def _sc_stage1(uid_hbm, mid_hbm, ut_hbm, vt_hbm, uval_hbm, vval_hbm,
               idbuf, lists, nlist, chunk, sval, sidx, val_sp,
               ssem, gsem, dsem, isem):
    c = lax.axis_index("c")
    s = lax.axis_index("s")
    iota16 = lax.iota(jnp.int32, 16)

    @pl.when(c == 0)
    def _():
        pltpu.make_async_copy(uid_hbm, idbuf, isem).start()

    @pl.when(c == 1)
    def _():
        pltpu.make_async_copy(mid_hbm, idbuf, isem).start()

    pltpu.make_async_copy(uid_hbm, idbuf, isem).wait()

    lo0 = s * _SEG
    hi0 = jnp.minimum(lo0 + _SEG, _SPLIT)
    lo1 = _SPLIT + s * _SEG
    hi1 = jnp.minimum(lo1 + _SEG, 1000000)

    def build(half, lo, hi):
        lov = jnp.full((16,), 1, jnp.int32) * lo
        hiv = jnp.full((16,), 1, jnp.int32) * hi

        def step(j, cur):
            off = pl.multiple_of(j * 16, 16)
            ids = idbuf[pl.ds(off, 16)]
            mask = (ids >= lov) & (ids < hiv)
            packed = (ids - lov) | ((iota16 + off) << 15)
            plsc.store_compressed(lists.at[half].at[pl.ds(cur, 16)], packed,
                                  mask=mask)
            cnt = plsc.all_reduce_population_count(mask)
            return cur + cnt[0]

        n = lax.fori_loop(0, _B // 16, step, 0)
        nlist[pl.ds(half * 16, 16)] = jnp.full((16,), 1, jnp.int32) * n

    build(0, lo0, hi0)
    build(1, lo1, hi1)

    def stream(p, slot):
        h = p & 1
        k = p >> 1
        src = pl.ds(h * _SPLIT + s * _SEG, _SEG)

        @pl.when((c == 0) & (p < _PHASES))
        def _():
            pltpu.make_async_copy(ut_hbm.at[pl.ds(k, 1), src], chunk.at[slot],
                                  ssem.at[slot]).start()

        @pl.when((c == 1) & (p < _PHASES))
        def _():
            pltpu.make_async_copy(vt_hbm.at[pl.ds(k, 1), src], chunk.at[slot],
                                  ssem.at[slot]).start()

    def stream_wait(slot):
        pltpu.make_async_copy(ut_hbm.at[pl.ds(0, 1), pl.ds(0, _SEG)],
                              chunk.at[slot], ssem.at[slot]).wait()

    def extract(p, slot):
        h = p & 1
        k = p >> 1
        vbase = (k & 1) * _VCAP
        nvec = nlist[pl.ds(pl.multiple_of(h * 16, 16), 16)]
        ngroups = jnp.maximum((nvec[0] + 511) >> 9, 1)
        slotvec = jnp.full((16,), 1, jnp.int32) * slot
        zerov = jnp.zeros((16,), jnp.int32)
        vbasev = jnp.full((16,), 1, jnp.int32) * vbase

        def block(g, ring):
            boff = pl.multiple_of(g * 512, 128) + ring * 128
            for v in range(8):
                goff = boff + v * 16
                lm = (iota16 + goff) < nvec
                pk = lists[h, pl.ds(goff, 16)]
                loc = jnp.where(lm, pk & 32767, 0)
                bb = jnp.where(lm, pk >> 15, _B) + vbasev
                vals = plsc.load_gather(chunk, [slotvec, zerov, loc])
                sval[ring, pl.ds(v * 16, 16)] = vals
                sidx[ring, pl.ds(v * 16, 16)] = bb
            pltpu.make_async_copy(
                sval.at[ring], val_sp.at[sidx.at[ring]], gsem.at[ring]).start()

        def group0(g, carry):
            for r in range(4):
                block(g, r)
            return carry

        def group(g, carry):
            for r in range(4):
                pltpu.make_async_copy(
                    sval.at[r], val_sp.at[sidx.at[r]], gsem.at[r]).wait()
                block(g, r)
            return carry

        lax.fori_loop(0, 1, group0, 0)
        lax.fori_loop(1, ngroups, group, 0)

        for r in range(4):
            pltpu.make_async_copy(
                sval.at[r], val_sp.at[sidx.at[r]], gsem.at[r]).wait()

    def dump(k):
        vbase = (k & 1) * _VCAP
        src = pl.ds(vbase + s * 1024, 1024)
        dst = pl.ds(pl.multiple_of(s * 1024, 8), 1024)

        @pl.when(c == 0)
        def _():
            pltpu.make_async_copy(val_sp.at[src], uval_hbm.at[k, dst],
                                  dsem).start()

        @pl.when(c == 1)
        def _():
            pltpu.make_async_copy(val_sp.at[src], vval_hbm.at[k, dst],
                                  dsem).start()

        pltpu.make_async_copy(val_sp.at[src], uval_hbm.at[k, dst], dsem).wait()

    stream(0, 0)

    def one_phase(p, slot):
        stream(p + 1, 1 - slot)
        stream_wait(slot)
        extract(p, slot)
        plsc.subcore_barrier()
        dump(p >> 1)

    def phase_pair(t, carry):
        one_phase(t * 2, 0)
        one_phase(t * 2 + 1, 1)
        return carry

    lax.fori_loop(0, _PHASES // 2, phase_pair, 0)


def _tc_dot(u_ref, v_ref, o_ref):
    o_ref[...] = jnp.sum(u_ref[...] * v_ref[...], axis=0)


@jax.jit
def kernel(uid, mid, u_emb, i_emb):
    mesh = plsc.VectorSubcoreMesh(core_axis_name="c", subcore_axis_name="s")
    stage1 = functools.partial(
        pl.kernel,
        mesh=mesh,
        out_type=(jax.ShapeDtypeStruct((_K, _B), jnp.float32),
                  jax.ShapeDtypeStruct((_K, _B), jnp.float32)),
        scratch_types=[
            pltpu.VMEM((_B,), jnp.int32),
            pltpu.VMEM((2, _LCAP), jnp.int32),
            pltpu.VMEM((32,), jnp.int32),
            pltpu.VMEM((2, 1, _SEG), jnp.float32),
            pltpu.VMEM((4, 128), jnp.float32),
            pltpu.VMEM((4, 128), jnp.int32),
            pltpu.VMEM_SHARED((2 * _VCAP,), jnp.float32),
            pltpu.SemaphoreType.DMA((2,)),
            pltpu.SemaphoreType.DMA((4,)),
            pltpu.SemaphoreType.DMA,
            pltpu.SemaphoreType.DMA,
        ],
        compiler_params=pltpu.CompilerParams(
            use_tc_tiling_on_sc=False, needs_layout_passes=False),
    )(_sc_stage1)
    uval, vval = stage1(uid, mid, u_emb.T, i_emb.T)

    out = pl.pallas_call(
        _tc_dot,
        out_shape=jax.ShapeDtypeStruct((_B,), jnp.float32),
        grid=(16,),
        in_specs=[
            pl.BlockSpec((_K, _B // 16), lambda j: (0, j)),
            pl.BlockSpec((_K, _B // 16), lambda j: (0, j)),
        ],
        out_specs=pl.BlockSpec((_B // 16,), lambda j: (j,)),
    )(uval, vval)
    return out


# speedup vs baseline: 9.2931x; 9.2931x over previous
import functools

import jax
import jax.numpy as jnp
from jax import lax
from jax.experimental import pallas as pl
from jax.experimental.pallas import tpu as pltpu
from jax.experimental.pallas import tpu_sc as plsc

_B = 16384
_K = 64
_NW = 32
_RPW = _B // _NW
_CHUNK = 128
_NCHUNK = _RPW // _CHUNK


def _sc_gather(idx_hbm, tab_hbm, out_hbm, idxv, rows, sem):
    wid = lax.axis_index("s") * 2 + lax.axis_index("c")

    pltpu.sync_copy(idx_hbm.at[wid], idxv)

    copies = []
    for c in range(_NCHUNK):
        dst = pl.ds(c * _CHUNK, _CHUNK)
        copies.append(pltpu.async_copy(tab_hbm.at[idxv.at[c]], rows.at[dst], sem))
    for cp in copies:
        cp.wait()

    pltpu.sync_copy(rows, out_hbm.at[wid])


def _tc_dot(u_ref, v_ref, o_ref):
    o_ref[...] = jnp.sum(u_ref[...] * v_ref[...], axis=-1).reshape(2 * _RPW)


def _gather(idx, tab):
    mesh = plsc.VectorSubcoreMesh(core_axis_name="c", subcore_axis_name="s")
    fn = functools.partial(
        pl.kernel,
        mesh=mesh,
        out_type=jax.ShapeDtypeStruct((_NW, _RPW, _K), jnp.float32),
        scratch_types=[
            pltpu.VMEM((_NCHUNK, _CHUNK), jnp.int32),
            pltpu.VMEM((_RPW, _K), jnp.float32),
            pltpu.SemaphoreType.DMA,
        ],
        compiler_params=pltpu.CompilerParams(use_tc_tiling_on_sc=False),
    )(_sc_gather)
    return fn(idx.reshape(_NW, _NCHUNK, _CHUNK), tab)


@jax.jit
def kernel(uid, mid, u_emb, i_emb):
    urows = _gather(uid, u_emb)
    vrows = _gather(mid, i_emb)

    out = pl.pallas_call(
        _tc_dot,
        out_shape=jax.ShapeDtypeStruct((_B,), jnp.float32),
        grid=(16,),
        in_specs=[
            pl.BlockSpec((2, _RPW, _K), lambda j: (j, 0, 0)),
            pl.BlockSpec((2, _RPW, _K), lambda j: (j, 0, 0)),
        ],
        out_specs=pl.BlockSpec((_B // 16,), lambda j: (j,)),
    )(urows, vrows)
    return out


# speedup vs baseline: 9.5071x; 1.0230x over previous
import functools

import jax
import jax.numpy as jnp
from jax import lax
from jax.experimental import pallas as pl
from jax.experimental.pallas import tpu as pltpu
from jax.experimental.pallas import tpu_sc as plsc

_B = 16384
_K = 64
_NW = 32
_RPW = _B // _NW
_CHUNK = 128
_NCHUNK = _RPW // _CHUNK
_GROUPS = _RPW // 16


def _sc_dot(uid_hbm, mid_hbm, u_emb_hbm, i_emb_hbm, out_hbm,
            uidx, midx, urows, vrows, outv, sem):
    wid = lax.axis_index("s") * 2 + lax.axis_index("c")

    pltpu.sync_copy(uid_hbm.at[wid], uidx)
    pltpu.sync_copy(mid_hbm.at[wid], midx)

    copies = []
    for c in range(_NCHUNK):
        dst = pl.ds(c * _CHUNK, _CHUNK)
        copies.append(pltpu.async_copy(u_emb_hbm.at[uidx.at[c]], urows.at[dst], sem))
        copies.append(pltpu.async_copy(i_emb_hbm.at[midx.at[c]], vrows.at[dst], sem))
    for cp in copies:
        cp.wait()

    iota16 = lax.iota(jnp.int32, 16)
    _dnums = lax.GatherDimensionNumbers(
        offset_dims=(), collapsed_slice_dims=(0,), start_index_map=(0,))

    def _shuffle(v, idx):
        return lax.gather(v, idx[:, None], _dnums, slice_sizes=(1,),
                          mode=lax.GatherScatterMode.PROMISE_IN_BOUNDS)

    def group(g, carry):
        off = pl.multiple_of(g * 16, 16)
        outvec = jnp.zeros((16,), jnp.float32)
        for rr in range(16):
            r = off + rr
            acc = jnp.zeros((16,), jnp.float32)
            for j in range(_K // 16):
                sl = pl.ds(j * 16, 16)
                acc = acc + urows[r, sl] * vrows[r, sl]
            for sh in (8, 4, 2, 1):
                acc = acc + _shuffle(acc, iota16 ^ sh)
            outvec = jnp.where(iota16 == rr, acc, outvec)
        outv[pl.ds(off, 16)] = outvec
        return carry

    lax.fori_loop(0, _GROUPS, group, 0)

    pltpu.sync_copy(outv, out_hbm.at[pl.ds(wid * _RPW, _RPW)])


@jax.jit
def kernel(uid, mid, u_emb, i_emb):
    mesh = plsc.VectorSubcoreMesh(core_axis_name="c", subcore_axis_name="s")
    fn = functools.partial(
        pl.kernel,
        mesh=mesh,
        out_type=jax.ShapeDtypeStruct((_B,), jnp.float32),
        scratch_types=[
            pltpu.VMEM((_NCHUNK, _CHUNK), jnp.int32),
            pltpu.VMEM((_NCHUNK, _CHUNK), jnp.int32),
            pltpu.VMEM((_RPW, _K), jnp.float32),
            pltpu.VMEM((_RPW, _K), jnp.float32),
            pltpu.VMEM((_RPW,), jnp.float32),
            pltpu.SemaphoreType.DMA,
        ],
        compiler_params=pltpu.CompilerParams(use_tc_tiling_on_sc=False),
    )(_sc_dot)
    return fn(uid.reshape(_NW, _NCHUNK, _CHUNK), mid.reshape(_NW, _NCHUNK, _CHUNK),
              u_emb, i_emb)
